# baseline (speedup 1.0000x reference)
import jax
import jax.numpy as jnp
from jax.experimental import pallas as pl
from jax.experimental.pallas import tpu as pltpu

_EPS = 1e-5
_NDOM = 4
_DPAD = 8


def _fwd_kernel(ids_ref, x_ref, pnw_ref, pnb_ref,
                cW1_ref, cb1_ref, cW2_ref, cb2_ref, cW3_ref, cb3_ref,
                dW1_ref, db1_ref, dW2_ref, db2_ref, dW3_ref, db3_ref,
                fW1_ref, fb1_ref, fW2_ref, fb2_ref,
                demb_ref, aW1_ref, ab1_ref, aW2_ref, ab2_ref,
                out_ref):
    f32 = jnp.float32
    bf16 = jnp.bfloat16
    bb = x_ref.shape[0]

    def dot(a, b):
        return jnp.dot(a, b, preferred_element_type=f32)

    x = x_ref[...]
    mean = jnp.mean(x, axis=1, keepdims=True)
    xc = x - mean
    var = jnp.mean(xc * xc, axis=1, keepdims=True)
    norm = xc * jax.lax.rsqrt(var + _EPS)

    ids = ids_ref[...]
    mask = ids == jax.lax.broadcasted_iota(jnp.int32, (bb, _DPAD), 1)
    onehot = jnp.where(mask, jnp.float32(1.0), jnp.float32(0.0))
    oh_b = onehot.astype(bf16)
    gamma = dot(oh_b, pnw_ref[...])
    beta = dot(oh_b, pnb_ref[...])
    normed = (norm * gamma + beta).astype(bf16)

    h = jax.nn.relu(dot(normed, cW1_ref[...]) + cb1_ref[...]).astype(bf16)
    h = jax.nn.relu(dot(h, cW2_ref[...]) + cb2_ref[...]).astype(bf16)
    h_center = dot(h, cW3_ref[...]) + cb3_ref[...]

    h_domain = None
    for d in range(_NDOM):
        t = jax.nn.relu(dot(normed, dW1_ref[d]) + db1_ref[d:d + 1, :])
        t = jax.nn.relu(dot(t.astype(bf16), dW2_ref[d]) + db2_ref[d:d + 1, :])
        t = dot(t.astype(bf16), dW3_ref[d]) + db3_ref[d:d + 1, :]
        t = onehot[:, d:d + 1] * t
        h_domain = t if h_domain is None else h_domain + t

    fused = h_center * jnp.tanh(h_domain)

    mp = jax.nn.relu(dot(fused.astype(bf16), fW1_ref[...]) + fb1_ref[...])
    main = dot(mp.astype(bf16), fW2_ref[...]) + fb2_ref[...]

    atab = jax.nn.relu(dot(demb_ref[...], aW1_ref[...]) + ab1_ref[...])
    atab = dot(atab.astype(bf16), aW2_ref[...]) + ab2_ref[...]
    aux = dot(oh_b, atab.astype(bf16))

    out_ref[...] = jax.nn.sigmoid(main + aux)


def kernel(x, domain_ids, pn_w, pn_b, cW1, cb1, cW2, cb2, cW3, cb3,
           dW1, db1, dW2, db2, dW3, db3, fW1, fb1, fW2, fb2,
           dom_emb, aW1, ab1, aW2, ab2):
    B, D_IN = x.shape
    BB = 512
    NB = B // BB
    f32 = jnp.float32

    def padrows(t):
        return jnp.pad(t, ((0, _DPAD - t.shape[0]), (0, 0)))

    ids8 = jnp.broadcast_to(domain_ids.astype(jnp.int32)[:, None], (B, _DPAD))
    bf = lambda t: t.astype(jnp.bfloat16)
    pnw8, pnb8, demb8 = bf(padrows(pn_w)), bf(padrows(pn_b)), bf(padrows(dom_emb))
    row = lambda v: v.reshape(1, -1).astype(f32)

    full = lambda t: pl.BlockSpec(t.shape, lambda i: (0,) * t.ndim)
    operands = [
        ids8, x, pnw8, pnb8,
        bf(cW1), row(cb1), bf(cW2), row(cb2), bf(cW3), row(cb3),
        bf(dW1), db1, bf(dW2), db2, bf(dW3), db3,
        bf(fW1), row(fb1), bf(fW2), row(fb2),
        demb8, bf(aW1), row(ab1), bf(aW2), row(ab2),
    ]
    in_specs = [
        pl.BlockSpec((BB, _DPAD), lambda i: (i, 0)),
        pl.BlockSpec((BB, D_IN), lambda i: (i, 0)),
    ] + [full(t) for t in operands[2:]]

    return pl.pallas_call(
        _fwd_kernel,
        grid=(NB,),
        in_specs=in_specs,
        out_specs=pl.BlockSpec((BB, 1), lambda i: (i, 0)),
        out_shape=jax.ShapeDtypeStruct((B, 1), f32),
        compiler_params=pltpu.CompilerParams(
            dimension_semantics=("parallel",),
            vmem_limit_bytes=50 * 1024 * 1024,
        ),
        name="hc2star_fused",
    )(*operands)


# speedup vs baseline: 1.1245x; 1.1245x over previous
import jax
import jax.numpy as jnp
from jax.experimental import pallas as pl
from jax.experimental.pallas import tpu as pltpu

_EPS = 1e-5
_NDOM = 4
_DPAD = 8


def _fwd_kernel(ids_ref, x_ref, pnwb_ref,
                W1_ref, b1_ref, cW2_ref, cb2_ref, cW3_ref, cb3_ref,
                dW2_ref, db2_ref, W3_ref, db3_ref,
                fW1_ref, fb1_ref, fW2_ref, fb2_ref,
                demb_ref, aW1_ref, ab1_ref, aW2_ref, ab2_ref,
                out_ref):
    f32 = jnp.float32
    bf16 = jnp.bfloat16
    bb = x_ref.shape[0]
    d_in = x_ref.shape[1]
    h1w = cW2_ref.shape[0]

    def dot(a, b):
        return jnp.dot(a, b, preferred_element_type=f32)

    x = x_ref[...]
    mean = jnp.mean(x, axis=1, keepdims=True)
    xc = x - mean
    var = jnp.mean(xc * xc, axis=1, keepdims=True)
    norm = xc * jax.lax.rsqrt(var + _EPS)

    ids = ids_ref[...]
    mask = ids == jax.lax.broadcasted_iota(jnp.int32, (bb, _DPAD), 1)
    onehot = jnp.where(mask, jnp.float32(1.0), jnp.float32(0.0))
    oh_b = onehot.astype(bf16)
    gb = dot(oh_b, pnwb_ref[...])
    normed = (norm * gb[:, :d_in] + gb[:, d_in:]).astype(bf16)

    h1 = jax.nn.relu(dot(normed, W1_ref[...]) + b1_ref[...])
    h1 = h1.astype(bf16)

    hc = jax.nn.relu(dot(h1[:, :h1w], cW2_ref[...]) + cb2_ref[...])
    h_center = dot(hc.astype(bf16), cW3_ref[...]) + cb3_ref[...]

    parts = []
    for d in range(_NDOM):
        sl = h1[:, (d + 1) * h1w:(d + 2) * h1w]
        t2 = jax.nn.relu(dot(sl, dW2_ref[d]) + db2_ref[d:d + 1, :])
        parts.append((onehot[:, d:d + 1] * t2).astype(bf16))
    u = jnp.concatenate(parts, axis=1)
    h_domain = dot(u, W3_ref[...]) + dot(oh_b, db3_ref[...])

    fused = h_center * jnp.tanh(h_domain)

    mp = jax.nn.relu(dot(fused.astype(bf16), fW1_ref[...]) + fb1_ref[...])
    main = dot(mp.astype(bf16), fW2_ref[...]) + fb2_ref[...]

    atab = jax.nn.relu(dot(demb_ref[...], aW1_ref[...]) + ab1_ref[...])
    atab = dot(atab.astype(bf16), aW2_ref[...]) + ab2_ref[...]
    aux = dot(oh_b, atab.astype(bf16))

    out_ref[...] = jax.nn.sigmoid(main + aux)


def kernel(x, domain_ids, pn_w, pn_b, cW1, cb1, cW2, cb2, cW3, cb3,
           dW1, db1, dW2, db2, dW3, db3, fW1, fb1, fW2, fb2,
           dom_emb, aW1, ab1, aW2, ab2):
    B, D_IN = x.shape
    BB = 512
    NB = B // BB
    f32 = jnp.float32
    bf = lambda t: t.astype(jnp.bfloat16)

    def padrows(t):
        return jnp.pad(t, ((0, _DPAD - t.shape[0]), (0, 0)))

    ids8 = jnp.broadcast_to(domain_ids.astype(jnp.int32)[:, None], (B, _DPAD))
    pnwb8 = bf(padrows(jnp.concatenate([pn_w, pn_b], axis=1)))
    demb8 = bf(padrows(dom_emb))
    W1cat = bf(jnp.concatenate(
        [cW1] + [dW1[d] for d in range(_NDOM)], axis=1))
    b1cat = jnp.concatenate(
        [cb1] + [db1[d] for d in range(_NDOM)]).reshape(1, -1).astype(f32)
    W3cat = bf(jnp.concatenate([dW3[d] for d in range(_NDOM)], axis=0))
    db38 = bf(padrows(db3))
    row = lambda v: v.reshape(1, -1).astype(f32)

    full = lambda t: pl.BlockSpec(t.shape, lambda i: (0,) * t.ndim)
    operands = [
        ids8, x, pnwb8,
        W1cat, b1cat, bf(cW2), row(cb2), bf(cW3), row(cb3),
        bf(dW2), db2, W3cat, db38,
        bf(fW1), row(fb1), bf(fW2), row(fb2),
        demb8, bf(aW1), row(ab1), bf(aW2), row(ab2),
    ]
    in_specs = [
        pl.BlockSpec((BB, _DPAD), lambda i: (i, 0)),
        pl.BlockSpec((BB, D_IN), lambda i: (i, 0)),
    ] + [full(t) for t in operands[2:]]

    return pl.pallas_call(
        _fwd_kernel,
        grid=(NB,),
        in_specs=in_specs,
        out_specs=pl.BlockSpec((BB, 1), lambda i: (i, 0)),
        out_shape=jax.ShapeDtypeStruct((B, 1), f32),
        compiler_params=pltpu.CompilerParams(
            dimension_semantics=("parallel",),
            vmem_limit_bytes=50 * 1024 * 1024,
        ),
        name="hc2star_fused",
    )(*operands)


# speedup vs baseline: 1.1811x; 1.0503x over previous
import jax
import jax.numpy as jnp
from jax.experimental import pallas as pl
from jax.experimental.pallas import tpu as pltpu

_EPS = 1e-5
_NDOM = 4
_DPAD = 8


def _fwd_kernel(ids_ref, x_ref, pnwb_ref,
                W1_ref, b1_ref, cW2_ref, cb2_ref, cW3_ref, cb3_ref,
                dW2_ref, db2_ref, W3_ref, db3_ref,
                fW1_ref, fb1_ref, fW2_ref, fb2_ref,
                demb_ref, aW1_ref, ab1_ref, aW2_ref, ab2_ref,
                out_ref):
    f32 = jnp.float32
    bf16 = jnp.bfloat16
    bb = x_ref.shape[0]
    d_in = x_ref.shape[1]
    h1w = cW2_ref.shape[0]

    def dot(a, b):
        return jnp.dot(a, b, preferred_element_type=f32)

    x = x_ref[...]
    mean = jnp.mean(x, axis=1, keepdims=True)
    xc = x - mean
    var = jnp.mean(xc * xc, axis=1, keepdims=True)
    norm = xc * jax.lax.rsqrt(var + _EPS)

    ids = ids_ref[...]
    mask = ids == jax.lax.broadcasted_iota(jnp.int32, (bb, _DPAD), 1)
    onehot = jnp.where(mask, jnp.float32(1.0), jnp.float32(0.0))
    oh_b = onehot.astype(bf16)
    gb = dot(oh_b, pnwb_ref[...])
    normed = (norm * gb[:, :d_in] + gb[:, d_in:]).astype(bf16)

    h1 = jax.nn.relu(dot(normed, W1_ref[...]) + b1_ref[...])
    h1 = h1.astype(bf16)

    hc = jax.nn.relu(dot(h1[:, :h1w], cW2_ref[...]) + cb2_ref[...])
    h_center = dot(hc.astype(bf16), cW3_ref[...]) + cb3_ref[...]

    parts = []
    for d in range(_NDOM):
        sl = h1[:, (d + 1) * h1w:(d + 2) * h1w]
        t2 = jax.nn.relu(dot(sl, dW2_ref[d]) + db2_ref[d:d + 1, :])
        parts.append((onehot[:, d:d + 1] * t2).astype(bf16))
    u = jnp.concatenate(parts, axis=1)
    h_domain = dot(u, W3_ref[...]) + dot(oh_b, db3_ref[...])

    fused = h_center * jnp.tanh(h_domain)

    mp = jax.nn.relu(dot(fused.astype(bf16), fW1_ref[...]) + fb1_ref[...])
    main = dot(mp.astype(bf16), fW2_ref[...]) + fb2_ref[...]

    atab = jax.nn.relu(dot(demb_ref[...], aW1_ref[...]) + ab1_ref[...])
    atab = dot(atab.astype(bf16), aW2_ref[...]) + ab2_ref[...]
    aux = dot(oh_b, atab.astype(bf16))

    out_ref[...] = jax.nn.sigmoid(main + aux)


def kernel(x, domain_ids, pn_w, pn_b, cW1, cb1, cW2, cb2, cW3, cb3,
           dW1, db1, dW2, db2, dW3, db3, fW1, fb1, fW2, fb2,
           dom_emb, aW1, ab1, aW2, ab2):
    B, D_IN = x.shape
    BB = 1024
    NB = B // BB
    f32 = jnp.float32
    bf = lambda t: t.astype(jnp.bfloat16)

    def padrows(t):
        return jnp.pad(t, ((0, _DPAD - t.shape[0]), (0, 0)))

    ids8 = jnp.broadcast_to(domain_ids.astype(jnp.int32)[:, None], (B, _DPAD))
    pnwb8 = bf(padrows(jnp.concatenate([pn_w, pn_b], axis=1)))
    demb8 = bf(padrows(dom_emb))
    W1cat = bf(jnp.concatenate(
        [cW1] + [dW1[d] for d in range(_NDOM)], axis=1))
    b1cat = jnp.concatenate(
        [cb1] + [db1[d] for d in range(_NDOM)]).reshape(1, -1).astype(f32)
    W3cat = bf(jnp.concatenate([dW3[d] for d in range(_NDOM)], axis=0))
    db38 = bf(padrows(db3))
    row = lambda v: v.reshape(1, -1).astype(f32)

    full = lambda t: pl.BlockSpec(t.shape, lambda i: (0,) * t.ndim)
    operands = [
        ids8, x, pnwb8,
        W1cat, b1cat, bf(cW2), row(cb2), bf(cW3), row(cb3),
        bf(dW2), db2, W3cat, db38,
        bf(fW1), row(fb1), bf(fW2), row(fb2),
        demb8, bf(aW1), row(ab1), bf(aW2), row(ab2),
    ]
    in_specs = [
        pl.BlockSpec((BB, _DPAD), lambda i: (i, 0)),
        pl.BlockSpec((BB, D_IN), lambda i: (i, 0)),
    ] + [full(t) for t in operands[2:]]

    return pl.pallas_call(
        _fwd_kernel,
        grid=(NB,),
        in_specs=in_specs,
        out_specs=pl.BlockSpec((BB, 1), lambda i: (i, 0)),
        out_shape=jax.ShapeDtypeStruct((B, 1), f32),
        compiler_params=pltpu.CompilerParams(
            dimension_semantics=("parallel",),
            vmem_limit_bytes=50 * 1024 * 1024,
        ),
        name="hc2star_fused",
    )(*operands)


# speedup vs baseline: 1.1852x; 1.0035x over previous
import jax
import jax.numpy as jnp
from jax.experimental import pallas as pl
from jax.experimental.pallas import tpu as pltpu

_EPS = 1e-5
_NDOM = 4
_DPAD = 8


def _fwd_kernel(ids_ref, x_ref, pnwb_ref,
                W1_ref, b1_ref, cW2_ref, cb2_ref, cW3_ref, cb3_ref,
                dW2_ref, db2_ref, W3_ref, db3_ref,
                fW1_ref, fb1_ref, fW2_ref, fb2_ref,
                demb_ref, aW1_ref, ab1_ref, aW2_ref, ab2_ref,
                out_ref):
    f32 = jnp.float32
    bf16 = jnp.bfloat16
    bb = x_ref.shape[0]
    d_in = x_ref.shape[1]
    h1w = cW2_ref.shape[0]

    def dot(a, b):
        return jnp.dot(a, b, preferred_element_type=f32)

    x = x_ref[...]
    mean = jnp.mean(x, axis=1, keepdims=True)
    xc = x - mean
    var = jnp.mean(xc * xc, axis=1, keepdims=True)
    norm = xc * jax.lax.rsqrt(var + _EPS)

    ids = ids_ref[...]
    mask = ids == jax.lax.broadcasted_iota(jnp.int32, (bb, _DPAD), 1)
    onehot = jnp.where(mask, jnp.float32(1.0), jnp.float32(0.0))
    oh_b = onehot.astype(bf16)
    gb = dot(oh_b, pnwb_ref[...])
    normed = (norm * gb[:, :d_in] + gb[:, d_in:]).astype(bf16)

    h1 = jax.nn.relu(dot(normed, W1_ref[...]) + b1_ref[...])
    h1 = h1.astype(bf16)

    hc = jax.nn.relu(dot(h1[:, :h1w], cW2_ref[...]) + cb2_ref[...])
    h_center = dot(hc.astype(bf16), cW3_ref[...]) + cb3_ref[...]

    parts = []
    for d in range(_NDOM):
        sl = h1[:, (d + 1) * h1w:(d + 2) * h1w]
        t2 = jax.nn.relu(dot(sl, dW2_ref[d]) + db2_ref[d:d + 1, :])
        parts.append((onehot[:, d:d + 1] * t2).astype(bf16))
    u = jnp.concatenate(parts, axis=1)
    h_domain = dot(u, W3_ref[...]) + dot(oh_b, db3_ref[...])

    fused = h_center * jnp.tanh(h_domain)

    mp = jax.nn.relu(dot(fused.astype(bf16), fW1_ref[...]) + fb1_ref[...])
    main = dot(mp.astype(bf16), fW2_ref[...]) + fb2_ref[...]

    atab = jax.nn.relu(dot(demb_ref[...], aW1_ref[...]) + ab1_ref[...])
    atab = dot(atab.astype(bf16), aW2_ref[...]) + ab2_ref[...]
    aux = dot(oh_b, atab.astype(bf16))

    out_ref[...] = jax.nn.sigmoid(main + aux)


def kernel(x, domain_ids, pn_w, pn_b, cW1, cb1, cW2, cb2, cW3, cb3,
           dW1, db1, dW2, db2, dW3, db3, fW1, fb1, fW2, fb2,
           dom_emb, aW1, ab1, aW2, ab2):
    B, D_IN = x.shape
    BB = 2048
    NB = B // BB
    f32 = jnp.float32
    bf = lambda t: t.astype(jnp.bfloat16)

    def padrows(t):
        return jnp.pad(t, ((0, _DPAD - t.shape[0]), (0, 0)))

    ids8 = jnp.broadcast_to(domain_ids.astype(jnp.int32)[:, None], (B, _DPAD))
    pnwb8 = bf(padrows(jnp.concatenate([pn_w, pn_b], axis=1)))
    demb8 = bf(padrows(dom_emb))
    W1cat = bf(jnp.concatenate(
        [cW1] + [dW1[d] for d in range(_NDOM)], axis=1))
    b1cat = jnp.concatenate(
        [cb1] + [db1[d] for d in range(_NDOM)]).reshape(1, -1).astype(f32)
    W3cat = bf(jnp.concatenate([dW3[d] for d in range(_NDOM)], axis=0))
    db38 = bf(padrows(db3))
    row = lambda v: v.reshape(1, -1).astype(f32)

    full = lambda t: pl.BlockSpec(t.shape, lambda i: (0,) * t.ndim)
    operands = [
        ids8, x, pnwb8,
        W1cat, b1cat, bf(cW2), row(cb2), bf(cW3), row(cb3),
        bf(dW2), db2, W3cat, db38,
        bf(fW1), row(fb1), bf(fW2), row(fb2),
        demb8, bf(aW1), row(ab1), bf(aW2), row(ab2),
    ]
    in_specs = [
        pl.BlockSpec((BB, _DPAD), lambda i: (i, 0)),
        pl.BlockSpec((BB, D_IN), lambda i: (i, 0)),
    ] + [full(t) for t in operands[2:]]

    return pl.pallas_call(
        _fwd_kernel,
        grid=(NB,),
        in_specs=in_specs,
        out_specs=pl.BlockSpec((BB, 1), lambda i: (i, 0)),
        out_shape=jax.ShapeDtypeStruct((B, 1), f32),
        compiler_params=pltpu.CompilerParams(
            dimension_semantics=("parallel",),
            vmem_limit_bytes=50 * 1024 * 1024,
        ),
        name="hc2star_fused",
    )(*operands)
